# baseline (speedup 1.0000x reference)
import jax
import jax.numpy as jnp
import numpy as np
from jax.experimental import pallas as pl

_SEQ_BLK = 1024
_STEP = 64
_KPAD = 128


def _pe_add_body(ts_ref, x_ref, hicat_ref, locat_ref, o_ref):
    s = pl.program_id(1)
    m = _SEQ_BLK
    d = x_ref.shape[-1]
    min_t = jnp.min(ts_ref[...])
    t_blk = ts_ref[0, 0, pl.ds(s * m, m)]
    delta = jnp.clip(t_blk - min_t, 0, jnp.int32(4999))
    hi = delta // _STEP
    lo = delta - hi * _STEP
    kio = jax.lax.broadcasted_iota(jnp.int32, (m, _KPAD), 1)
    a = (hi[:, None] == kio).astype(jnp.bfloat16)
    b = (lo[:, None] == kio).astype(jnp.bfloat16)
    hsw = jnp.dot(a, hicat_ref[...], preferred_element_type=jnp.float32)
    uv = jnp.dot(b, locat_ref[...], preferred_element_type=jnp.float32)
    pe = hsw[:, :d] * uv[:, :d] + hsw[:, d:] * uv[:, d:]
    o_ref[0, :, :] = x_ref[0, :, :] + pe


def _pair_swap(t):
    n, d = t.shape
    return t.reshape(n, d // 2, 2)[:, :, ::-1].reshape(n, d)


def kernel(x, timesteps, pos_encoding):
    b, seq, one, d = x.shape

    x3 = x.reshape(b, seq, d)
    ts = timesteps.reshape(b, 1, seq).astype(jnp.int32)

    n_hi = (pos_encoding.shape[0] + _STEP - 1) // _STEP
    hi_t = pos_encoding[:: _STEP]
    lo_t = pos_encoding[:_STEP]
    hi_sw = _pair_swap(hi_t)
    lo_sw = _pair_swap(lo_t)
    even = (jnp.arange(d) % 2 == 0)[None, :]
    u_t = jnp.where(even, lo_sw, lo_t)
    v_t = jnp.where(even, lo_t, -lo_sw)

    def _pad(t, rows):
        return jnp.pad(t, ((0, _KPAD - rows), (0, 0)))

    hicat = jnp.concatenate([_pad(hi_t, n_hi), _pad(hi_sw, n_hi)], axis=1)
    locat = jnp.concatenate([_pad(u_t, _STEP), _pad(v_t, _STEP)], axis=1)
    hicat = hicat.astype(jnp.bfloat16)
    locat = locat.astype(jnp.bfloat16)

    n_s = seq // _SEQ_BLK
    out = pl.pallas_call(
        _pe_add_body,
        grid=(b, n_s),
        in_specs=[
            pl.BlockSpec((1, 1, seq), lambda i, j: (i, 0, 0)),
            pl.BlockSpec((1, _SEQ_BLK, d), lambda i, j: (i, j, 0)),
            pl.BlockSpec((_KPAD, 2 * d), lambda i, j: (0, 0)),
            pl.BlockSpec((_KPAD, 2 * d), lambda i, j: (0, 0)),
        ],
        out_specs=pl.BlockSpec((1, _SEQ_BLK, d), lambda i, j: (i, j, 0)),
        out_shape=jax.ShapeDtypeStruct((b, seq, d), x.dtype),
    )(ts, x3, hicat, locat)
    return out.reshape(b, seq, one, d)


# speedup vs baseline: 1.0386x; 1.0386x over previous
import jax
import jax.numpy as jnp
import numpy as np
from jax.experimental import pallas as pl

_SEQ_BLK = 2048
_STEP = 64
_KPAD = 128


def _pe_add_body(ts_ref, x_ref, hicat_ref, locat_ref, o_ref):
    s = pl.program_id(1)
    m = _SEQ_BLK
    d = x_ref.shape[-1]
    min_t = jnp.min(ts_ref[...])
    t_blk = ts_ref[0, 0, pl.ds(s * m, m)]
    delta = jnp.clip(t_blk - min_t, 0, jnp.int32(4999))
    hi = delta // _STEP
    lo = delta - hi * _STEP
    kio = jax.lax.broadcasted_iota(jnp.int32, (m, _KPAD), 1)
    a = (hi[:, None] == kio).astype(jnp.bfloat16)
    b = (lo[:, None] == kio).astype(jnp.bfloat16)
    hsw = jnp.dot(a, hicat_ref[...], preferred_element_type=jnp.float32)
    uv = jnp.dot(b, locat_ref[...], preferred_element_type=jnp.float32)
    pe = hsw[:, :d] * uv[:, :d] + hsw[:, d:] * uv[:, d:]
    o_ref[0, :, :] = x_ref[0, :, :] + pe


def _pair_swap(t):
    n, d = t.shape
    return t.reshape(n, d // 2, 2)[:, :, ::-1].reshape(n, d)


def kernel(x, timesteps, pos_encoding):
    b, seq, one, d = x.shape

    x3 = x.reshape(b, seq, d)
    ts = timesteps.reshape(b, 1, seq).astype(jnp.int32)

    n_hi = (pos_encoding.shape[0] + _STEP - 1) // _STEP
    hi_t = pos_encoding[:: _STEP]
    lo_t = pos_encoding[:_STEP]
    hi_sw = _pair_swap(hi_t)
    lo_sw = _pair_swap(lo_t)
    even = (jnp.arange(d) % 2 == 0)[None, :]
    u_t = jnp.where(even, lo_sw, lo_t)
    v_t = jnp.where(even, lo_t, -lo_sw)

    def _pad(t, rows):
        return jnp.pad(t, ((0, _KPAD - rows), (0, 0)))

    hicat = jnp.concatenate([_pad(hi_t, n_hi), _pad(hi_sw, n_hi)], axis=1)
    locat = jnp.concatenate([_pad(u_t, _STEP), _pad(v_t, _STEP)], axis=1)
    hicat = hicat.astype(jnp.bfloat16)
    locat = locat.astype(jnp.bfloat16)

    n_s = seq // _SEQ_BLK
    out = pl.pallas_call(
        _pe_add_body,
        grid=(b, n_s),
        in_specs=[
            pl.BlockSpec((1, 1, seq), lambda i, j: (i, 0, 0)),
            pl.BlockSpec((1, _SEQ_BLK, d), lambda i, j: (i, j, 0)),
            pl.BlockSpec((_KPAD, 2 * d), lambda i, j: (0, 0)),
            pl.BlockSpec((_KPAD, 2 * d), lambda i, j: (0, 0)),
        ],
        out_specs=pl.BlockSpec((1, _SEQ_BLK, d), lambda i, j: (i, j, 0)),
        out_shape=jax.ShapeDtypeStruct((b, seq, d), x.dtype),
    )(ts, x3, hicat, locat)
    return out.reshape(b, seq, one, d)
